# baseline (device time: 148232 ns/iter reference)
import jax
import jax.numpy as jnp
from jax import lax
from jax.experimental import pallas as pl
from jax.experimental.pallas import tpu as pltpu

N_DEV = 4
T = 1024
D = 1024
F = 2048
E = 16
E_LOCAL = E // N_DEV
T_SHARD = T // N_DEV
H = T_SHARD // 2
FBLK = 512


def _mesh_copy(src, dst, send_sem, recv_sem, dev):
    return pltpu.make_async_remote_copy(
        src_ref=src, dst_ref=dst, send_sem=send_sem, recv_sem=recv_sem,
        device_id=(dev,), device_id_type=pl.DeviceIdType.MESH,
    )


def _neighbor_barrier(left, right):
    barrier = pltpu.get_barrier_semaphore()
    for nbr in (left, right):
        pl.semaphore_signal(barrier, inc=1, device_id=(nbr,),
                            device_id_type=pl.DeviceIdType.MESH)
    pl.semaphore_wait(barrier, 2)


def _rag_body(r_ref, rout_ref, send_sems, recv_sems):
    my = lax.axis_index("i")
    left = lax.rem(my + N_DEV - 1, N_DEV)
    right = lax.rem(my + 1, N_DEV)
    _neighbor_barrier(left, right)

    rout_ref[my] = r_ref[...]
    for h in range(N_DEV - 1):
        origin = lax.rem(my + N_DEV - h, N_DEV)
        rdma = _mesh_copy(rout_ref.at[origin], rout_ref.at[origin],
                          send_sems.at[h], recv_sems.at[h], right)
        rdma.start()
        rdma.wait()


def _router_allgather(router_t):
    return pl.pallas_call(
        _rag_body,
        out_shape=jax.ShapeDtypeStruct((N_DEV, E_LOCAL, D), jnp.float32),
        in_specs=[pl.BlockSpec(memory_space=pltpu.VMEM)],
        out_specs=pl.BlockSpec(memory_space=pltpu.VMEM),
        scratch_shapes=[
            pltpu.SemaphoreType.DMA((N_DEV - 1,)),
            pltpu.SemaphoreType.DMA((N_DEV - 1,)),
        ],
        compiler_params=pltpu.CompilerParams(collective_id=0),
    )(router_t)


def _fused_body(x_ref, w_ref, w1_ref, w2_ref, out_ref, cx, cw, ssem, rsem):
    e = pl.program_id(0)
    f = pl.program_id(1)
    c = pl.program_id(2)
    my = lax.axis_index("i")
    left = lax.rem(my + N_DEV - 1, N_DEV)
    right = lax.rem(my + 1, N_DEV)

    def ch(i, src, dst, dev):
        return _mesh_copy(src, dst, ssem.at[i], rsem.at[i], dev)

    def ch_x_h1r():
        return ch(0, cx.at[0], cx.at[1], right)

    def ch_x_h1l():
        return ch(1, cx.at[0], cx.at[2], left)

    def ch_w_h1r():
        return ch(2, cw.at[0], cw.at[1], right)

    def ch_w_h1l():
        return ch(3, cw.at[0], cw.at[2], left)

    def ch_x_fwr():
        return ch(4, cx.at[1, pl.ds(0, H)], cx.at[3, pl.ds(0, H)], right)

    def ch_x_fwl():
        return ch(5, cx.at[2, pl.ds(H, H)], cx.at[3, pl.ds(H, H)], left)

    def ch_w_fwr():
        return ch(6, cw.at[1, pl.ds(0, H)], cw.at[3, pl.ds(0, H)], right)

    def ch_w_fwl():
        return ch(7, cw.at[2, pl.ds(H, H)], cw.at[3, pl.ds(H, H)], left)

    first_ef = jnp.logical_and(e == 0, f == 0)

    @pl.when(jnp.logical_and(first_ef, c == 0))
    def _():
        _neighbor_barrier(left, right)
        cx[0] = x_ref[...]
        cw[0] = w_ref[...]
        ch_x_h1r().start()
        ch_x_h1l().start()
        ch_w_h1r().start()
        ch_w_h1l().start()

    @pl.when(jnp.logical_and(first_ef, c == 1))
    def _():
        ch_x_h1r().wait_recv()
        ch_w_h1r().wait_recv()
        ch_x_fwr().start()
        ch_w_fwr().start()

    @pl.when(jnp.logical_and(first_ef, c == 2))
    def _():
        ch_x_h1l().wait_recv()
        ch_w_h1l().wait_recv()
        ch_x_fwl().start()
        ch_w_fwl().start()

    @pl.when(jnp.logical_and(first_ef, c == 3))
    def _():
        ch_x_fwr().wait_recv()
        ch_x_fwl().wait_recv()
        ch_w_fwr().wait_recv()
        ch_w_fwl().wait_recv()

    @pl.when(first_ef)
    def _():
        out_ref[c] = jnp.zeros((T_SHARD, D), jnp.float32)

    h = jnp.maximum(
        jnp.dot(cx[c], w1_ref[0], preferred_element_type=jnp.float32), 0.0)
    contrib = jnp.dot(h, w2_ref[0], preferred_element_type=jnp.float32)
    g = my * E_LOCAL + e
    onehot = (lax.broadcasted_iota(jnp.int32, (1, E), 1) == g).astype(
        jnp.float32)
    wcol = jnp.sum(cw[c] * onehot, axis=1, keepdims=True)
    out_ref[c] += contrib * wcol

    @pl.when(jnp.logical_and(jnp.logical_and(e == E_LOCAL - 1,
                                             f == F // FBLK - 1),
                             c == N_DEV - 1))
    def _():
        ch_x_h1r().wait_send()
        ch_x_h1l().wait_send()
        ch_w_h1r().wait_send()
        ch_w_h1l().wait_send()
        ch_x_fwr().wait_send()
        ch_x_fwl().wait_send()
        ch_w_fwr().wait_send()
        ch_w_fwl().wait_send()


def _fused_ag_compute(x_shard, w_my, W1, W2):
    return pl.pallas_call(
        _fused_body,
        grid=(E_LOCAL, F // FBLK, N_DEV),
        in_specs=[
            pl.BlockSpec(memory_space=pltpu.VMEM),
            pl.BlockSpec(memory_space=pltpu.VMEM),
            pl.BlockSpec((1, D, FBLK), lambda e, f, c: (e, 0, f)),
            pl.BlockSpec((1, FBLK, D), lambda e, f, c: (e, f, 0)),
        ],
        out_specs=pl.BlockSpec((N_DEV, T_SHARD, D), lambda e, f, c: (0, 0, 0)),
        out_shape=jax.ShapeDtypeStruct((N_DEV, T_SHARD, D), jnp.float32),
        scratch_shapes=[
            pltpu.VMEM((N_DEV, T_SHARD, D), jnp.float32),
            pltpu.VMEM((N_DEV, T_SHARD, E), jnp.float32),
            pltpu.SemaphoreType.DMA((8,)),
            pltpu.SemaphoreType.DMA((8,)),
        ],
        compiler_params=pltpu.CompilerParams(
            collective_id=1,
            dimension_semantics=("arbitrary", "arbitrary", "arbitrary"),
        ),
    )(x_shard, w_my, W1, W2)


def _rs_body(part_ref, out_ref, dirL, dirR, trL, trR, dhalf, ssem, rsem):
    my = lax.axis_index("i")
    left = lax.rem(my + N_DEV - 1, N_DEV)
    right = lax.rem(my + 1, N_DEV)
    _neighbor_barrier(left, right)

    def ch(i, src, dst, dev):
        return _mesh_copy(src, dst, ssem.at[i], rsem.at[i], dev)

    c0 = ch(0, part_ref.at[2], dirL, right)
    c1 = ch(1, part_ref.at[1], dirR, left)
    c2 = ch(2, part_ref.at[3, pl.ds(0, H)], trL, right)
    c3 = ch(3, part_ref.at[3, pl.ds(H, H)], trR, left)
    c0.start()
    c1.start()
    c2.start()
    c3.start()

    c2.wait_recv()
    c4 = ch(4, trL, dhalf.at[pl.ds(0, H)], right)
    c4.start()
    c3.wait_recv()
    c5 = ch(5, trR, dhalf.at[pl.ds(H, H)], left)
    c5.start()

    c0.wait_recv()
    c1.wait_recv()
    c4.wait_recv()
    c5.wait_recv()
    out_ref[...] = part_ref[0] + dirL[...] + dirR[...] + dhalf[...]

    for c in (c0, c1, c2, c3, c4, c5):
        c.wait_send()


def _reduce_scatter(part):
    return pl.pallas_call(
        _rs_body,
        out_shape=jax.ShapeDtypeStruct((T_SHARD, D), jnp.float32),
        in_specs=[pl.BlockSpec(memory_space=pltpu.VMEM)],
        out_specs=pl.BlockSpec(memory_space=pltpu.VMEM),
        scratch_shapes=[
            pltpu.VMEM((T_SHARD, D), jnp.float32),
            pltpu.VMEM((T_SHARD, D), jnp.float32),
            pltpu.VMEM((H, D), jnp.float32),
            pltpu.VMEM((H, D), jnp.float32),
            pltpu.VMEM((T_SHARD, D), jnp.float32),
            pltpu.SemaphoreType.DMA((6,)),
            pltpu.SemaphoreType.DMA((6,)),
        ],
        compiler_params=pltpu.CompilerParams(collective_id=2),
    )(part)


def kernel(x, router, W1, W2):
    rout = _router_allgather(router.T)
    router_t_full = rout.reshape(E, D)

    gates = jnp.einsum("td,ed->te", x, router_t_full,
                       precision=lax.Precision.HIGHEST)
    top_vals, top_idx = lax.top_k(gates, 2)
    p2 = jnp.exp(top_vals[:, 1] - top_vals[:, 0])
    w1 = 1.0 / (1.0 + p2)
    w2 = p2 / (1.0 + p2)
    eids = jnp.arange(E, dtype=top_idx.dtype)
    w_my = ((top_idx[:, 0:1] == eids[None, :]) * w1[:, None]
            + (top_idx[:, 1:2] == eids[None, :]) * w2[:, None])
    w_my = w_my.astype(jnp.float32)

    part = _fused_ag_compute(x, w_my, W1, W2)

    return _reduce_scatter(part)


# device time: 81842 ns/iter; 1.8112x vs baseline; 1.8112x over previous
import jax
import jax.numpy as jnp
from jax import lax
from jax.experimental import pallas as pl
from jax.experimental.pallas import tpu as pltpu

N_DEV = 4
T = 1024
D = 1024
F = 2048
E = 16
E_LOCAL = E // N_DEV
TS = T // N_DEV
H = TS // 2
FBLK = 1024
BF16 = jnp.bfloat16


def _mesh_copy(src, dst, send_sem, recv_sem, dev):
    return pltpu.make_async_remote_copy(
        src_ref=src, dst_ref=dst, send_sem=send_sem, recv_sem=recv_sem,
        device_id=(dev,), device_id_type=pl.DeviceIdType.MESH,
    )


def _neighbor_barrier(left, right):
    barrier = pltpu.get_barrier_semaphore()
    for nbr in (left, right):
        pl.semaphore_signal(barrier, inc=1, device_id=(nbr,),
                            device_id_type=pl.DeviceIdType.MESH)
    pl.semaphore_wait(barrier, 2)


def _onehot(k):
    return (lax.broadcasted_iota(jnp.int32, (1, E), 1) == k).astype(
        jnp.float32)


def _fused_body(x_ref, r_ref, w1_ref, w2_ref, out_ref,
                cx, cw, rout, acc, ssem, rsem, rss, rsr):
    e = pl.program_id(0)
    f = pl.program_id(1)
    my = lax.axis_index("i")
    left = lax.rem(my + N_DEV - 1, N_DEV)
    right = lax.rem(my + 1, N_DEV)

    def ch(i, ref, s0, d0, n, dev):
        return _mesh_copy(ref.at[pl.ds(s0, n)], ref.at[pl.ds(d0, n)],
                          ssem.at[i], rsem.at[i], dev)

    def ch_x_h1r():
        return ch(0, cx, 0, TS, TS, right)

    def ch_x_h1l():
        return ch(1, cx, 0, 2 * TS, TS, left)

    def ch_w_h1r():
        return ch(2, cw, 0, TS, TS, right)

    def ch_w_h1l():
        return ch(3, cw, 0, 2 * TS, TS, left)

    def ch_x_fwr():
        return ch(4, cx, TS, 3 * TS, H, right)

    def ch_x_fwl():
        return ch(5, cx, 3 * TS - H, 3 * TS + H, H, left)

    def ch_w_fwr():
        return ch(6, cw, TS, 3 * TS, H, right)

    def ch_w_fwl():
        return ch(7, cw, 3 * TS - H, 3 * TS + H, H, left)

    def rch(i, s0, d0, dev):
        return _mesh_copy(rout.at[pl.ds(s0, E_LOCAL)],
                          rout.at[pl.ds(d0, E_LOCAL)],
                          rss.at[i], rsr.at[i], dev)

    def rch0():
        return rch(0, 0, E_LOCAL, right)

    def rch1():
        return rch(1, 0, 2 * E_LOCAL, left)

    def rch2():
        return rch(2, E_LOCAL, 3 * E_LOCAL, right)

    first_ef = jnp.logical_and(e == 0, f == 0)
    last_ef = jnp.logical_and(e == E_LOCAL - 1, f == F // FBLK - 1)

    @pl.when(first_ef)
    def _():
        _neighbor_barrier(left, right)
        cx[0:TS] = x_ref[...].astype(BF16)
        ch_x_h1r().start()
        ch_x_h1l().start()
        rout[0:E_LOCAL] = r_ref[...]
        rch0().start()
        rch1().start()
        rch0().wait_recv()
        rch2().start()
        rch1().wait_recv()
        rch2().wait_recv()
        gates = lax.dot_general(
            x_ref[...], rout[...], (((1,), (1,)), ((), ())),
            precision=lax.Precision.HIGHEST,
            preferred_element_type=jnp.float32,
        )
        m1 = jnp.max(gates, axis=1, keepdims=True)
        is1 = (gates == m1).astype(jnp.float32)
        masked = jnp.where(gates == m1, -1e30, gates)
        m2 = jnp.max(masked, axis=1, keepdims=True)
        is2 = (masked == m2).astype(jnp.float32)
        p2 = jnp.exp(m2 - m1)
        wa = 1.0 / (1.0 + p2)
        wb = p2 / (1.0 + p2)
        cw[0:TS] = is1 * wa + is2 * wb
        ch_w_h1r().start()
        ch_w_h1l().start()
        ch_x_h1r().wait_recv()
        ch_x_fwr().start()
        ch_x_h1l().wait_recv()
        ch_x_fwl().start()
        ch_w_h1r().wait_recv()
        ch_w_fwr().start()
        ch_w_h1l().wait_recv()
        ch_w_fwl().start()
        ch_x_fwr().wait_recv()
        ch_x_fwl().wait_recv()
        ch_w_fwr().wait_recv()
        ch_w_fwl().wait_recv()
        acc[...] = jnp.zeros((T, D), jnp.float32)

    w1b = w1_ref[0].astype(BF16)
    w2b = w2_ref[0].astype(BF16)
    h = jnp.maximum(
        jnp.dot(cx[...], w1b, preferred_element_type=jnp.float32),
        0.0).astype(BF16)
    contrib = jnp.dot(h, w2b, preferred_element_type=jnp.float32)
    w_own = jnp.sum(cw[0:TS] * _onehot(e), axis=1, keepdims=True)
    w_l = jnp.sum(cw[TS:2 * TS] * _onehot(2 * E_LOCAL + e),
                  axis=1, keepdims=True)
    w_r = jnp.sum(cw[2 * TS:3 * TS] * _onehot(E_LOCAL + e),
                  axis=1, keepdims=True)
    w_d = jnp.sum(cw[3 * TS:T] * _onehot(3 * E_LOCAL + e),
                  axis=1, keepdims=True)
    wcol = jnp.concatenate([w_own, w_l, w_r, w_d], axis=0)
    acc[...] += contrib * wcol

    @pl.when(last_ef)
    def _():
        out_ref[...] = acc[...].astype(BF16)
        for mk in (ch_x_h1r, ch_x_h1l, ch_w_h1r, ch_w_h1l,
                   ch_x_fwr, ch_x_fwl, ch_w_fwr, ch_w_fwl,
                   rch0, rch1, rch2):
            mk().wait_send()


def _fused_all(x_shard, router_t, W1, W2):
    return pl.pallas_call(
        _fused_body,
        grid=(E_LOCAL, F // FBLK),
        in_specs=[
            pl.BlockSpec(memory_space=pltpu.VMEM),
            pl.BlockSpec(memory_space=pltpu.VMEM),
            pl.BlockSpec((1, D, FBLK), lambda e, f: (e, 0, f)),
            pl.BlockSpec((1, FBLK, D), lambda e, f: (e, f, 0)),
        ],
        out_specs=pl.BlockSpec((T, D), lambda e, f: (0, 0)),
        out_shape=jax.ShapeDtypeStruct((T, D), BF16),
        scratch_shapes=[
            pltpu.VMEM((T, D), BF16),
            pltpu.VMEM((T, E), jnp.float32),
            pltpu.VMEM((E, D), jnp.float32),
            pltpu.VMEM((T, D), jnp.float32),
            pltpu.SemaphoreType.DMA((8,)),
            pltpu.SemaphoreType.DMA((8,)),
            pltpu.SemaphoreType.DMA((3,)),
            pltpu.SemaphoreType.DMA((3,)),
        ],
        compiler_params=pltpu.CompilerParams(
            collective_id=1,
            dimension_semantics=("arbitrary", "arbitrary"),
        ),
    )(x_shard, router_t, W1, W2)


def _rs_body(part_ref, out_ref, dirL, dirR, trL, trR, dhalf, ssem, rsem):
    my = lax.axis_index("i")
    left = lax.rem(my + N_DEV - 1, N_DEV)
    right = lax.rem(my + 1, N_DEV)
    _neighbor_barrier(left, right)

    def ch(i, src, dst, dev):
        return _mesh_copy(src, dst, ssem.at[i], rsem.at[i], dev)

    c0 = ch(0, part_ref.at[2], dirL, right)
    c1 = ch(1, part_ref.at[1], dirR, left)
    c2 = ch(2, part_ref.at[3, pl.ds(0, H)], trL, right)
    c3 = ch(3, part_ref.at[3, pl.ds(H, H)], trR, left)
    c0.start()
    c1.start()
    c2.start()
    c3.start()

    c2.wait_recv()
    c4 = ch(4, trL, dhalf.at[pl.ds(0, H)], right)
    c4.start()
    c3.wait_recv()
    c5 = ch(5, trR, dhalf.at[pl.ds(H, H)], left)
    c5.start()

    c0.wait_recv()
    c1.wait_recv()
    c4.wait_recv()
    c5.wait_recv()
    out_ref[...] = (part_ref[0].astype(jnp.float32)
                    + dirL[...].astype(jnp.float32)
                    + dirR[...].astype(jnp.float32)
                    + dhalf[...].astype(jnp.float32))

    for c in (c0, c1, c2, c3, c4, c5):
        c.wait_send()


def _reduce_scatter(part):
    return pl.pallas_call(
        _rs_body,
        out_shape=jax.ShapeDtypeStruct((TS, D), jnp.float32),
        in_specs=[pl.BlockSpec(memory_space=pltpu.VMEM)],
        out_specs=pl.BlockSpec(memory_space=pltpu.VMEM),
        scratch_shapes=[
            pltpu.VMEM((TS, D), BF16),
            pltpu.VMEM((TS, D), BF16),
            pltpu.VMEM((H, D), BF16),
            pltpu.VMEM((H, D), BF16),
            pltpu.VMEM((TS, D), BF16),
            pltpu.SemaphoreType.DMA((6,)),
            pltpu.SemaphoreType.DMA((6,)),
        ],
        compiler_params=pltpu.CompilerParams(collective_id=2),
    )(part)


def kernel(x, router, W1, W2):
    part = _fused_all(x, router.T, W1, W2)
    return _reduce_scatter(part.reshape(N_DEV, TS, D))


# device time: 68434 ns/iter; 2.1661x vs baseline; 1.1959x over previous
import jax
import jax.numpy as jnp
from jax import lax
from jax.experimental import pallas as pl
from jax.experimental.pallas import tpu as pltpu

N_DEV = 4
T = 1024
D = 1024
F = 2048
E = 16
E_LOCAL = E // N_DEV
TS = T // N_DEV
H = TS // 2
C = 192
FBLK = 1024
BF16 = jnp.bfloat16


def _mesh_copy(src, dst, send_sem, recv_sem, dev):
    return pltpu.make_async_remote_copy(
        src_ref=src, dst_ref=dst, send_sem=send_sem, recv_sem=recv_sem,
        device_id=(dev,), device_id_type=pl.DeviceIdType.MESH,
    )


def _neighbor_barrier(left, right):
    barrier = pltpu.get_barrier_semaphore()
    for nbr in (left, right):
        pl.semaphore_signal(barrier, inc=1, device_id=(nbr,),
                            device_id_type=pl.DeviceIdType.MESH)
    pl.semaphore_wait(barrier, 2)


def _ag_body(x_ref, r_ref, cx, cw, rout, ssem, rsem, rss, rsr):
    my = lax.axis_index("i")
    left = lax.rem(my + N_DEV - 1, N_DEV)
    right = lax.rem(my + 1, N_DEV)

    def ch(i, ref, s0, d0, n, dev):
        return _mesh_copy(ref.at[pl.ds(s0, n)], ref.at[pl.ds(d0, n)],
                          ssem.at[i], rsem.at[i], dev)

    def ch_x_h1r():
        return ch(0, cx, 0, TS, TS, right)

    def ch_x_h1l():
        return ch(1, cx, 0, 2 * TS, TS, left)

    def ch_w_h1r():
        return ch(2, cw, 0, TS, TS, right)

    def ch_w_h1l():
        return ch(3, cw, 0, 2 * TS, TS, left)

    def ch_x_fwr():
        return ch(4, cx, TS, 3 * TS, H, right)

    def ch_x_fwl():
        return ch(5, cx, 3 * TS - H, 3 * TS + H, H, left)

    def ch_w_fwr():
        return ch(6, cw, TS, 3 * TS, H, right)

    def ch_w_fwl():
        return ch(7, cw, 3 * TS - H, 3 * TS + H, H, left)

    def rch(i, s0, d0, dev):
        return _mesh_copy(rout.at[pl.ds(s0, E_LOCAL)],
                          rout.at[pl.ds(d0, E_LOCAL)],
                          rss.at[i], rsr.at[i], dev)

    def rch0():
        return rch(0, 0, E_LOCAL, right)

    def rch1():
        return rch(1, 0, 2 * E_LOCAL, left)

    def rch2():
        return rch(2, E_LOCAL, 3 * E_LOCAL, right)

    _neighbor_barrier(left, right)
    cx[0:TS] = x_ref[...].astype(BF16)
    ch_x_h1r().start()
    ch_x_h1l().start()
    rout[0:E_LOCAL] = r_ref[...]
    rch0().start()
    rch1().start()
    rch0().wait_recv()
    rch2().start()
    rch1().wait_recv()
    rch2().wait_recv()
    gates = lax.dot_general(
        x_ref[...], rout[...], (((1,), (1,)), ((), ())),
        precision=lax.Precision.HIGHEST,
        preferred_element_type=jnp.float32,
    )
    m1 = jnp.max(gates, axis=1, keepdims=True)
    is1 = (gates == m1).astype(jnp.float32)
    masked = jnp.where(gates == m1, -1e30, gates)
    m2 = jnp.max(masked, axis=1, keepdims=True)
    is2 = (masked == m2).astype(jnp.float32)
    p2 = jnp.exp(m2 - m1)
    wa = 1.0 / (1.0 + p2)
    wb = p2 / (1.0 + p2)
    cw[0:TS] = is1 * wa + is2 * wb
    ch_w_h1r().start()
    ch_w_h1l().start()
    ch_x_h1r().wait_recv()
    ch_x_fwr().start()
    ch_x_h1l().wait_recv()
    ch_x_fwl().start()
    ch_w_h1r().wait_recv()
    ch_w_fwr().start()
    ch_w_h1l().wait_recv()
    ch_w_fwl().start()
    ch_x_fwr().wait_recv()
    ch_x_fwl().wait_recv()
    ch_w_fwr().wait_recv()
    ch_w_fwl().wait_recv()
    for mk in (ch_x_h1r, ch_x_h1l, ch_w_h1r, ch_w_h1l,
               ch_x_fwr, ch_x_fwl, ch_w_fwr, ch_w_fwl,
               rch0, rch1, rch2):
        mk().wait_send()


def _ag_call(x_shard, router_t):
    return pl.pallas_call(
        _ag_body,
        out_shape=(
            jax.ShapeDtypeStruct((T, D), BF16),
            jax.ShapeDtypeStruct((T, E), jnp.float32),
        ),
        in_specs=[
            pl.BlockSpec(memory_space=pltpu.VMEM),
            pl.BlockSpec(memory_space=pltpu.VMEM),
        ],
        out_specs=(
            pl.BlockSpec(memory_space=pltpu.VMEM),
            pl.BlockSpec(memory_space=pltpu.VMEM),
        ),
        scratch_shapes=[
            pltpu.VMEM((E, D), jnp.float32),
            pltpu.SemaphoreType.DMA((8,)),
            pltpu.SemaphoreType.DMA((8,)),
            pltpu.SemaphoreType.DMA((3,)),
            pltpu.SemaphoreType.DMA((3,)),
        ],
        compiler_params=pltpu.CompilerParams(collective_id=1),
    )(x_shard, router_t)


def _sparse_body(cx_ref, oneh_ref, onehT_ref, wl_ref, w1_ref, w2_ref,
                 out_ref, xg, wgs, yacc, acc,
                 pstage, dirL, dirR, trL, trR, dhalf, qss, qsr):
    e = pl.program_id(0)
    f = pl.program_id(1)
    n_f = F // FBLK
    my = lax.axis_index("i")
    left = lax.rem(my + N_DEV - 1, N_DEV)
    right = lax.rem(my + 1, N_DEV)

    @pl.when(jnp.logical_and(e == 0, f == 0))
    def _():
        _neighbor_barrier(left, right)

    @pl.when(f == 0)
    def _():
        xg[...] = jnp.dot(onehT_ref[e], cx_ref[...],
                          preferred_element_type=jnp.float32).astype(BF16)
        wsel = jnp.dot(onehT_ref[e].astype(jnp.float32), wl_ref[...],
                       preferred_element_type=jnp.float32)
        sel = (lax.broadcasted_iota(jnp.int32, (1, E_LOCAL), 1) == e
               ).astype(jnp.float32)
        wgs[...] = jnp.sum(wsel * sel, axis=1, keepdims=True)

    w1b = w1_ref[0].astype(BF16)
    w2b = w2_ref[0].astype(BF16)
    h = jnp.maximum(
        jnp.dot(xg[...], w1b, preferred_element_type=jnp.float32),
        0.0).astype(BF16)
    y = jnp.dot(h, w2b, preferred_element_type=jnp.float32)

    @pl.when(f == 0)
    def _():
        yacc[...] = y

    @pl.when(f != 0)
    def _():
        yacc[...] += y

    @pl.when(f == n_f - 1)
    def _():
        scaled = (yacc[...] * wgs[...]).astype(BF16)
        comb = jnp.dot(oneh_ref[e], scaled,
                       preferred_element_type=jnp.float32)

        @pl.when(e == 0)
        def _():
            acc[...] = comb

        @pl.when(e != 0)
        def _():
            acc[...] += comb

    def qch(i, src, dst, dev):
        return _mesh_copy(src, dst, qss.at[i], qsr.at[i], dev)

    def q_dir_r():
        return qch(0, pstage.at[pl.ds(TS, TS)], dirL, right)

    def q_dir_l():
        return qch(1, pstage.at[pl.ds(0, TS)], dirR, left)

    def q_half_r():
        return qch(2, pstage.at[pl.ds(2 * TS, H)], trL, right)

    def q_half_l():
        return qch(3, pstage.at[pl.ds(2 * TS + H, H)], trR, left)

    def q_fwd_r():
        return qch(4, trL, dhalf.at[pl.ds(0, H)], right)

    def q_fwd_l():
        return qch(5, trR, dhalf.at[pl.ds(H, H)], left)

    @pl.when(jnp.logical_and(e == E_LOCAL - 1, f == n_f - 1))
    def _():
        pstage[...] = acc[TS:T].astype(BF16)
        q_dir_r().start()
        q_dir_l().start()
        q_half_r().start()
        q_half_l().start()
        q_half_r().wait_recv()
        q_fwd_r().start()
        q_half_l().wait_recv()
        q_fwd_l().start()
        q_dir_r().wait_recv()
        q_dir_l().wait_recv()
        q_fwd_r().wait_recv()
        q_fwd_l().wait_recv()
        out_ref[...] = (acc[0:TS]
                        + dirL[...].astype(jnp.float32)
                        + dirR[...].astype(jnp.float32)
                        + dhalf[...].astype(jnp.float32))
        for mk in (q_dir_r, q_dir_l, q_half_r, q_half_l,
                   q_fwd_r, q_fwd_l):
            mk().wait_send()


def _sparse_call(cx, oneh, onehT, w_loc, W1, W2):
    return pl.pallas_call(
        _sparse_body,
        grid=(E_LOCAL, F // FBLK),
        in_specs=[
            pl.BlockSpec(memory_space=pltpu.VMEM),
            pl.BlockSpec(memory_space=pltpu.VMEM),
            pl.BlockSpec(memory_space=pltpu.VMEM),
            pl.BlockSpec(memory_space=pltpu.VMEM),
            pl.BlockSpec((1, D, FBLK), lambda e, f: (e, 0, f)),
            pl.BlockSpec((1, FBLK, D), lambda e, f: (e, f, 0)),
        ],
        out_specs=pl.BlockSpec((TS, D), lambda e, f: (0, 0)),
        out_shape=jax.ShapeDtypeStruct((TS, D), jnp.float32),
        scratch_shapes=[
            pltpu.VMEM((C, D), BF16),
            pltpu.VMEM((C, 1), jnp.float32),
            pltpu.VMEM((C, D), jnp.float32),
            pltpu.VMEM((T, D), jnp.float32),
            pltpu.VMEM((3 * TS, D), BF16),
            pltpu.VMEM((TS, D), BF16),
            pltpu.VMEM((TS, D), BF16),
            pltpu.VMEM((H, D), BF16),
            pltpu.VMEM((H, D), BF16),
            pltpu.VMEM((TS, D), BF16),
            pltpu.SemaphoreType.DMA((6,)),
            pltpu.SemaphoreType.DMA((6,)),
        ],
        compiler_params=pltpu.CompilerParams(
            collective_id=2,
            dimension_semantics=("arbitrary", "arbitrary"),
        ),
    )(cx, oneh, onehT, w_loc, W1, W2)


def _rs_body(part_ref, out_ref, dirL, dirR, trL, trR, dhalf, ssem, rsem):
    my = lax.axis_index("i")
    left = lax.rem(my + N_DEV - 1, N_DEV)
    right = lax.rem(my + 1, N_DEV)
    _neighbor_barrier(left, right)

    def ch(i, src, dst, dev):
        return _mesh_copy(src, dst, ssem.at[i], rsem.at[i], dev)

    c0 = ch(0, part_ref.at[2], dirL, right)
    c1 = ch(1, part_ref.at[1], dirR, left)
    c2 = ch(2, part_ref.at[3, pl.ds(0, H)], trL, right)
    c3 = ch(3, part_ref.at[3, pl.ds(H, H)], trR, left)
    c0.start()
    c1.start()
    c2.start()
    c3.start()

    c2.wait_recv()
    c4 = ch(4, trL, dhalf.at[pl.ds(0, H)], right)
    c4.start()
    c3.wait_recv()
    c5 = ch(5, trR, dhalf.at[pl.ds(H, H)], left)
    c5.start()

    c0.wait_recv()
    c1.wait_recv()
    c4.wait_recv()
    c5.wait_recv()
    out_ref[...] = (part_ref[0].astype(jnp.float32)
                    + dirL[...].astype(jnp.float32)
                    + dirR[...].astype(jnp.float32)
                    + dhalf[...].astype(jnp.float32))

    for c in (c0, c1, c2, c3, c4, c5):
        c.wait_send()


def _reduce_scatter(part):
    return pl.pallas_call(
        _rs_body,
        out_shape=jax.ShapeDtypeStruct((TS, D), jnp.float32),
        in_specs=[pl.BlockSpec(memory_space=pltpu.VMEM)],
        out_specs=pl.BlockSpec(memory_space=pltpu.VMEM),
        scratch_shapes=[
            pltpu.VMEM((TS, D), BF16),
            pltpu.VMEM((TS, D), BF16),
            pltpu.VMEM((H, D), BF16),
            pltpu.VMEM((H, D), BF16),
            pltpu.VMEM((TS, D), BF16),
            pltpu.SemaphoreType.DMA((6,)),
            pltpu.SemaphoreType.DMA((6,)),
        ],
        compiler_params=pltpu.CompilerParams(collective_id=2),
    )(part)


def kernel(x, router, W1, W2):
    cx, cw = _ag_call(x, router.T)

    w_loc = jnp.concatenate([
        cw[0:TS, 0:4], cw[TS:2 * TS, 8:12],
        cw[2 * TS:3 * TS, 4:8], cw[3 * TS:T, 12:16],
    ], axis=0)

    routed = w_loc > 0.0
    rank = jnp.cumsum(routed.astype(jnp.int32), axis=0) - 1
    rank = jnp.where(routed, rank, -1)
    slots = jnp.arange(C, dtype=jnp.int32)
    oneh = (rank.T[:, :, None] == slots[None, None, :]).astype(BF16)
    onehT = (rank.T[:, None, :] == slots[None, :, None]).astype(BF16)
    return _sparse_call(cx, oneh, onehT, w_loc, W1, W2)


# device time: 67425 ns/iter; 2.1985x vs baseline; 1.0150x over previous
import jax
import jax.numpy as jnp
from jax import lax
from jax.experimental import pallas as pl
from jax.experimental.pallas import tpu as pltpu

N_DEV = 4
T = 1024
D = 1024
F = 2048
E = 16
E_LOCAL = E // N_DEV
TS = T // N_DEV
H = TS // 2
C = 192
FBLK = 1024
BF16 = jnp.bfloat16


def _mesh_copy(src, dst, send_sem, recv_sem, dev):
    return pltpu.make_async_remote_copy(
        src_ref=src, dst_ref=dst, send_sem=send_sem, recv_sem=recv_sem,
        device_id=(dev,), device_id_type=pl.DeviceIdType.MESH,
    )


def _neighbor_barrier(left, right):
    barrier = pltpu.get_barrier_semaphore()
    for nbr in (left, right):
        pl.semaphore_signal(barrier, inc=1, device_id=(nbr,),
                            device_id_type=pl.DeviceIdType.MESH)
    pl.semaphore_wait(barrier, 2)


def _ag_body(x_ref, r_ref, cx, cw, rout, ssem, rsem, rss, rsr):
    my = lax.axis_index("i")
    left = lax.rem(my + N_DEV - 1, N_DEV)
    right = lax.rem(my + 1, N_DEV)

    def ch(i, ref, s0, d0, n, dev):
        return _mesh_copy(ref.at[pl.ds(s0, n)], ref.at[pl.ds(d0, n)],
                          ssem.at[i], rsem.at[i], dev)

    def ch_x_h1r():
        return ch(0, cx, 0, TS, TS, right)

    def ch_x_h1l():
        return ch(1, cx, 0, 2 * TS, TS, left)

    def ch_w_h1r():
        return ch(2, cw, 0, TS, TS, right)

    def ch_w_h1l():
        return ch(3, cw, 0, 2 * TS, TS, left)

    def ch_x_fwr():
        return ch(4, cx, TS, 3 * TS, H, right)

    def ch_x_fwl():
        return ch(5, cx, 3 * TS - H, 3 * TS + H, H, left)

    def ch_w_fwr():
        return ch(6, cw, TS, 3 * TS, H, right)

    def ch_w_fwl():
        return ch(7, cw, 3 * TS - H, 3 * TS + H, H, left)

    def rch(i, s0, d0, dev):
        return _mesh_copy(rout.at[pl.ds(s0, E_LOCAL)],
                          rout.at[pl.ds(d0, E_LOCAL)],
                          rss.at[i], rsr.at[i], dev)

    def rch0():
        return rch(0, 0, E_LOCAL, right)

    def rch1():
        return rch(1, 0, 2 * E_LOCAL, left)

    def rch2():
        return rch(2, 0, 3 * E_LOCAL, lax.rem(my + 2, N_DEV))

    _neighbor_barrier(left, right)
    cx[0:TS] = x_ref[...].astype(BF16)
    ch_x_h1r().start()
    ch_x_h1l().start()
    rout[0:E_LOCAL] = r_ref[...]
    rch0().start()
    rch1().start()
    rch2().start()
    rch0().wait_recv()
    rch1().wait_recv()
    rch2().wait_recv()
    gates = lax.dot_general(
        x_ref[...], rout[...], (((1,), (1,)), ((), ())),
        precision=lax.Precision.HIGHEST,
        preferred_element_type=jnp.float32,
    )
    m1 = jnp.max(gates, axis=1, keepdims=True)
    is1 = (gates == m1).astype(jnp.float32)
    masked = jnp.where(gates == m1, -1e30, gates)
    m2 = jnp.max(masked, axis=1, keepdims=True)
    is2 = (masked == m2).astype(jnp.float32)
    p2 = jnp.exp(m2 - m1)
    wa = 1.0 / (1.0 + p2)
    wb = p2 / (1.0 + p2)
    cw[0:TS] = is1 * wa + is2 * wb
    ch_w_h1r().start()
    ch_w_h1l().start()
    ch_x_h1r().wait_recv()
    ch_x_fwr().start()
    ch_x_h1l().wait_recv()
    ch_x_fwl().start()
    ch_w_h1r().wait_recv()
    ch_w_fwr().start()
    ch_w_h1l().wait_recv()
    ch_w_fwl().start()
    ch_x_fwr().wait_recv()
    ch_x_fwl().wait_recv()
    ch_w_fwr().wait_recv()
    ch_w_fwl().wait_recv()
    for mk in (ch_x_h1r, ch_x_h1l, ch_w_h1r, ch_w_h1l,
               ch_x_fwr, ch_x_fwl, ch_w_fwr, ch_w_fwl,
               rch0, rch1, rch2):
        mk().wait_send()


def _ag_call(x_shard, router_t):
    return pl.pallas_call(
        _ag_body,
        out_shape=(
            jax.ShapeDtypeStruct((T, D), BF16),
            jax.ShapeDtypeStruct((T, E), jnp.float32),
        ),
        in_specs=[
            pl.BlockSpec(memory_space=pltpu.VMEM),
            pl.BlockSpec(memory_space=pltpu.VMEM),
        ],
        out_specs=(
            pl.BlockSpec(memory_space=pltpu.VMEM),
            pl.BlockSpec(memory_space=pltpu.VMEM),
        ),
        scratch_shapes=[
            pltpu.VMEM((E, D), jnp.float32),
            pltpu.SemaphoreType.DMA((8,)),
            pltpu.SemaphoreType.DMA((8,)),
            pltpu.SemaphoreType.DMA((3,)),
            pltpu.SemaphoreType.DMA((3,)),
        ],
        compiler_params=pltpu.CompilerParams(collective_id=1),
    )(x_shard, router_t)


def _sparse_body(cx_ref, oneh_ref, onehT_ref, wl_ref, w1_ref, w2_ref,
                 out_ref, xg, wgs, yacc, acc,
                 pstage, dirL, dirR, trL, trR, dhalf, qss, qsr):
    e = pl.program_id(0)
    f = pl.program_id(1)
    n_f = F // FBLK
    my = lax.axis_index("i")
    left = lax.rem(my + N_DEV - 1, N_DEV)
    right = lax.rem(my + 1, N_DEV)

    @pl.when(jnp.logical_and(e == 0, f == 0))
    def _():
        _neighbor_barrier(left, right)

    @pl.when(f == 0)
    def _():
        xg[...] = jnp.dot(onehT_ref[e], cx_ref[...],
                          preferred_element_type=jnp.float32).astype(BF16)
        wsel = jnp.dot(onehT_ref[e].astype(jnp.float32), wl_ref[...],
                       preferred_element_type=jnp.float32)
        sel = (lax.broadcasted_iota(jnp.int32, (1, E_LOCAL), 1) == e
               ).astype(jnp.float32)
        wgs[...] = jnp.sum(wsel * sel, axis=1, keepdims=True)

    w1b = w1_ref[0].astype(BF16)
    w2b = w2_ref[0].astype(BF16)
    h = jnp.maximum(
        jnp.dot(xg[...], w1b, preferred_element_type=jnp.float32),
        0.0).astype(BF16)
    y = jnp.dot(h, w2b, preferred_element_type=jnp.float32)

    @pl.when(f == 0)
    def _():
        yacc[...] = y

    @pl.when(f != 0)
    def _():
        yacc[...] += y

    @pl.when(f == n_f - 1)
    def _():
        scaled = (yacc[...] * wgs[...]).astype(BF16)
        comb = jnp.dot(oneh_ref[e], scaled,
                       preferred_element_type=jnp.float32)

        @pl.when(e == 0)
        def _():
            acc[...] = comb

        @pl.when(e != 0)
        def _():
            acc[...] += comb

    def qch(i, src, dst, dev):
        return _mesh_copy(src, dst, qss.at[i], qsr.at[i], dev)

    def q_dir_r():
        return qch(0, pstage.at[pl.ds(TS, TS)], dirL, right)

    def q_dir_l():
        return qch(1, pstage.at[pl.ds(0, TS)], dirR, left)

    def q_half_r():
        return qch(2, pstage.at[pl.ds(2 * TS, H)], trL, right)

    def q_half_l():
        return qch(3, pstage.at[pl.ds(2 * TS + H, H)], trR, left)

    def q_fwd_r():
        return qch(4, trL, dhalf.at[pl.ds(0, H)], right)

    def q_fwd_l():
        return qch(5, trR, dhalf.at[pl.ds(H, H)], left)

    @pl.when(jnp.logical_and(e == E_LOCAL - 1, f == n_f - 1))
    def _():
        pstage[...] = acc[TS:T].astype(BF16)
        q_dir_r().start()
        q_dir_l().start()
        q_half_r().start()
        q_half_l().start()
        q_half_r().wait_recv()
        q_fwd_r().start()
        q_half_l().wait_recv()
        q_fwd_l().start()
        q_dir_r().wait_recv()
        q_dir_l().wait_recv()
        q_fwd_r().wait_recv()
        q_fwd_l().wait_recv()
        out_ref[...] = (acc[0:TS]
                        + dirL[...].astype(jnp.float32)
                        + dirR[...].astype(jnp.float32)
                        + dhalf[...].astype(jnp.float32))
        for mk in (q_dir_r, q_dir_l, q_half_r, q_half_l,
                   q_fwd_r, q_fwd_l):
            mk().wait_send()


def _sparse_call(cx, oneh, onehT, w_loc, W1, W2):
    return pl.pallas_call(
        _sparse_body,
        grid=(E_LOCAL, F // FBLK),
        in_specs=[
            pl.BlockSpec(memory_space=pltpu.VMEM),
            pl.BlockSpec(memory_space=pltpu.VMEM),
            pl.BlockSpec(memory_space=pltpu.VMEM),
            pl.BlockSpec(memory_space=pltpu.VMEM),
            pl.BlockSpec((1, D, FBLK), lambda e, f: (e, 0, f)),
            pl.BlockSpec((1, FBLK, D), lambda e, f: (e, f, 0)),
        ],
        out_specs=pl.BlockSpec((TS, D), lambda e, f: (0, 0)),
        out_shape=jax.ShapeDtypeStruct((TS, D), jnp.float32),
        scratch_shapes=[
            pltpu.VMEM((C, D), BF16),
            pltpu.VMEM((C, 1), jnp.float32),
            pltpu.VMEM((C, D), jnp.float32),
            pltpu.VMEM((T, D), jnp.float32),
            pltpu.VMEM((3 * TS, D), BF16),
            pltpu.VMEM((TS, D), BF16),
            pltpu.VMEM((TS, D), BF16),
            pltpu.VMEM((H, D), BF16),
            pltpu.VMEM((H, D), BF16),
            pltpu.VMEM((TS, D), BF16),
            pltpu.SemaphoreType.DMA((6,)),
            pltpu.SemaphoreType.DMA((6,)),
        ],
        compiler_params=pltpu.CompilerParams(
            collective_id=2,
            dimension_semantics=("arbitrary", "arbitrary"),
        ),
    )(cx, oneh, onehT, w_loc, W1, W2)


def _rs_body(part_ref, out_ref, dirL, dirR, trL, trR, dhalf, ssem, rsem):
    my = lax.axis_index("i")
    left = lax.rem(my + N_DEV - 1, N_DEV)
    right = lax.rem(my + 1, N_DEV)
    _neighbor_barrier(left, right)

    def ch(i, src, dst, dev):
        return _mesh_copy(src, dst, ssem.at[i], rsem.at[i], dev)

    c0 = ch(0, part_ref.at[2], dirL, right)
    c1 = ch(1, part_ref.at[1], dirR, left)
    c2 = ch(2, part_ref.at[3, pl.ds(0, H)], trL, right)
    c3 = ch(3, part_ref.at[3, pl.ds(H, H)], trR, left)
    c0.start()
    c1.start()
    c2.start()
    c3.start()

    c2.wait_recv()
    c4 = ch(4, trL, dhalf.at[pl.ds(0, H)], right)
    c4.start()
    c3.wait_recv()
    c5 = ch(5, trR, dhalf.at[pl.ds(H, H)], left)
    c5.start()

    c0.wait_recv()
    c1.wait_recv()
    c4.wait_recv()
    c5.wait_recv()
    out_ref[...] = (part_ref[0].astype(jnp.float32)
                    + dirL[...].astype(jnp.float32)
                    + dirR[...].astype(jnp.float32)
                    + dhalf[...].astype(jnp.float32))

    for c in (c0, c1, c2, c3, c4, c5):
        c.wait_send()


def _reduce_scatter(part):
    return pl.pallas_call(
        _rs_body,
        out_shape=jax.ShapeDtypeStruct((TS, D), jnp.float32),
        in_specs=[pl.BlockSpec(memory_space=pltpu.VMEM)],
        out_specs=pl.BlockSpec(memory_space=pltpu.VMEM),
        scratch_shapes=[
            pltpu.VMEM((TS, D), BF16),
            pltpu.VMEM((TS, D), BF16),
            pltpu.VMEM((H, D), BF16),
            pltpu.VMEM((H, D), BF16),
            pltpu.VMEM((TS, D), BF16),
            pltpu.SemaphoreType.DMA((6,)),
            pltpu.SemaphoreType.DMA((6,)),
        ],
        compiler_params=pltpu.CompilerParams(collective_id=2),
    )(part)


def kernel(x, router, W1, W2):
    cx, cw = _ag_call(x, router.T)

    w_loc = jnp.concatenate([
        cw[0:TS, 0:4], cw[TS:2 * TS, 8:12],
        cw[2 * TS:3 * TS, 4:8], cw[3 * TS:T, 12:16],
    ], axis=0)

    routed = w_loc > 0.0
    rank = jnp.cumsum(routed.astype(jnp.int32), axis=0) - 1
    rank = jnp.where(routed, rank, -1)
    slots = jnp.arange(C, dtype=jnp.int32)
    oneh = (rank.T[:, :, None] == slots[None, None, :]).astype(BF16)
    onehT = (rank.T[:, None, :] == slots[None, :, None]).astype(BF16)
    return _sparse_call(cx, oneh, onehT, w_loc, W1, W2)
